# baseline (speedup 1.0000x reference)
import functools

import jax
import jax.numpy as jnp
from jax import lax
from jax.experimental import pallas as pl
from jax.experimental.pallas import tpu as pltpu
from jax.experimental.pallas import tpu_sc as plsc

N = 10000
D = 128
E = 320000
G = 32

NC = 2
NS = 16
NW = NC * NS
CH = 128
J = 80
NP = 2
JP = J // NP
HP = JP // 2
E_PAD = NW * CH * J
ROWS_PER_TILE = 632
NPAD = NS * ROWS_PER_TILE

RBLK = 2000
GRID = N // RBLK


def _sc_body(hw_hbm, src_hbm, dst_hbm, zeros_hbm, out_hbm,
             srcv, dstv, rows_a, rows_b, agg, sem_a, sem_b, sem_sa, sem_sb):
    cid = lax.axis_index("c")
    sid = lax.axis_index("s")
    wid = sid * NC + cid

    pltpu.sync_copy(zeros_hbm, agg.at[pl.ds(sid * ROWS_PER_TILE, ROWS_PER_TILE)])
    plsc.subcore_barrier()

    def _fill(buf, sem, j):
        pltpu.async_copy(hw_hbm.at[srcv.at[j]], buf, sem)

    def _drain(buf, sem):
        pltpu.make_async_copy(hw_hbm.at[pl.ds(0, CH)], buf, sem).wait()

    def _scatter(buf, sem, j):
        pltpu.async_copy(buf, agg.at[dstv.at[j]], sem, add=True)

    def _drain_s(buf, sem):
        pltpu.make_async_copy(hw_hbm.at[pl.ds(0, CH)], buf, sem).wait()

    for p in range(NP):
        pltpu.sync_copy(src_hbm.at[pl.ds(wid * J + p * JP, JP)], srcv)
        pltpu.sync_copy(dst_hbm.at[pl.ds(wid * J + p * JP, JP)], dstv)
        _fill(rows_a, sem_a, 0)
        _fill(rows_b, sem_b, 1)

        def step(i, carry):
            _drain(rows_a, sem_a)
            _scatter(rows_a, sem_sa, 2 * i)
            _drain(rows_b, sem_b)
            _scatter(rows_b, sem_sb, 2 * i + 1)

            @pl.when(i < HP - 1)
            def _next():
                _drain_s(rows_a, sem_sa)
                _fill(rows_a, sem_a, 2 * i + 2)
                _drain_s(rows_b, sem_sb)
                _fill(rows_b, sem_b, 2 * i + 3)

            return carry

        lax.fori_loop(0, HP, step, 0)
        _drain_s(rows_a, sem_sa)
        _drain_s(rows_b, sem_sb)
    plsc.subcore_barrier()

    pltpu.sync_copy(agg.at[pl.ds(sid * ROWS_PER_TILE, ROWS_PER_TILE)],
                    out_hbm.at[cid, pl.ds(sid * ROWS_PER_TILE, ROWS_PER_TILE)])


_sc_scatter = functools.partial(
    pl.kernel,
    mesh=plsc.VectorSubcoreMesh(core_axis_name="c", subcore_axis_name="s",
                                num_cores=NC, num_subcores=NS),
    out_type=jax.ShapeDtypeStruct((NC, NPAD, D), jnp.float32),
    scratch_types=[
        pltpu.VMEM((JP, CH), jnp.int32),
        pltpu.VMEM((JP, CH), jnp.int32),
        pltpu.VMEM((CH, D), jnp.float32),
        pltpu.VMEM((CH, D), jnp.float32),
        pltpu.VMEM_SHARED((NPAD, D), jnp.float32),
        pltpu.SemaphoreType.DMA,
        pltpu.SemaphoreType.DMA,
        pltpu.SemaphoreType.DMA,
        pltpu.SemaphoreType.DMA,
    ],
)(_sc_body)


def _mm_body(x_ref, w_ref, o_ref):
    o_ref[...] = jnp.dot(x_ref[...], w_ref[...], preferred_element_type=jnp.float32)


_mm = pl.pallas_call(
    _mm_body,
    grid=(GRID,),
    in_specs=[
        pl.BlockSpec((RBLK, D), lambda i: (i, 0)),
        pl.BlockSpec((D, D), lambda i: (0, 0)),
    ],
    out_specs=pl.BlockSpec((RBLK, D), lambda i: (i, 0)),
    out_shape=jax.ShapeDtypeStruct((N, D), jnp.float32),
)


def _act_mm_body(p_ref, b_ref, w_ref, o_ref):
    h = jnp.maximum(p_ref[0] + p_ref[1] + b_ref[...], 0.0)
    o_ref[...] = jnp.dot(h, w_ref[...], preferred_element_type=jnp.float32)


_act_mm = pl.pallas_call(
    _act_mm_body,
    grid=(GRID,),
    in_specs=[
        pl.BlockSpec((NC, RBLK, D), lambda i: (0, i, 0)),
        pl.BlockSpec((1, D), lambda i: (0, 0)),
        pl.BlockSpec((D, D), lambda i: (0, 0)),
    ],
    out_specs=pl.BlockSpec((RBLK, D), lambda i: (i, 0)),
    out_shape=jax.ShapeDtypeStruct((N, D), jnp.float32),
)


def _pool_body(p_ref, b_ref, bat_ref, o_ref, sums, counts):
    i = pl.program_id(0)

    @pl.when(i == 0)
    def _init():
        sums[...] = jnp.zeros_like(sums)
        counts[...] = jnp.zeros_like(counts)

    h = jnp.maximum(p_ref[0] + p_ref[1] + b_ref[...], 0.0)
    onehot = (bat_ref[...] == lax.broadcasted_iota(jnp.int32, (1, G), 1)
              ).astype(jnp.float32)
    dn = (((0,), (0,)), ((), ()))
    sums[...] += lax.dot_general(onehot, h, dn, preferred_element_type=jnp.float32)
    counts[...] += lax.dot_general(onehot, jnp.ones_like(h), dn,
                                   preferred_element_type=jnp.float32)

    @pl.when(i == pl.num_programs(0) - 1)
    def _fin():
        o_ref[...] = sums[...] / jnp.maximum(counts[...], 1.0)


_pool = pl.pallas_call(
    _pool_body,
    grid=(GRID,),
    in_specs=[
        pl.BlockSpec((NC, RBLK, D), lambda i: (0, i, 0)),
        pl.BlockSpec((1, D), lambda i: (0, 0)),
        pl.BlockSpec((RBLK, 1), lambda i: (i, 0)),
    ],
    out_specs=pl.BlockSpec((G, D), lambda i: (0, 0)),
    out_shape=jax.ShapeDtypeStruct((G, D), jnp.float32),
    scratch_shapes=[
        pltpu.VMEM((G, D), jnp.float32),
        pltpu.VMEM((G, D), jnp.float32),
    ],
)


def kernel(x, edge_index, batch, W1, b1, W2, b2, W3, b3, W4, b4):
    src = edge_index[0].astype(jnp.int32)
    dst = edge_index[1].astype(jnp.int32)
    pad = E_PAD - E
    pad_src = jnp.arange(pad, dtype=jnp.int32) % N
    pad_dst = N + (jnp.arange(pad, dtype=jnp.int32) % (NPAD - N))
    src_r = jnp.concatenate([src, pad_src]).reshape(NW * J, CH)
    dst_r = jnp.concatenate([dst, pad_dst]).reshape(NW * J, CH)
    zeros = jnp.zeros((ROWS_PER_TILE, D), jnp.float32)
    bat = batch.astype(jnp.int32).reshape(N, 1)

    hw = _mm(x, W1)
    parts = _sc_scatter(hw, src_r, dst_r, zeros)
    for b, W in ((b1, W2), (b2, W3), (b3, W4)):
        hw = _act_mm(parts, b.reshape(1, D), W)
        parts = _sc_scatter(hw, src_r, dst_r, zeros)
    return _pool(parts, b4.reshape(1, D), bat)


# speedup vs baseline: 1.3439x; 1.3439x over previous
import functools

import jax
import jax.numpy as jnp
from jax import lax
from jax.experimental import pallas as pl
from jax.experimental.pallas import tpu as pltpu
from jax.experimental.pallas import tpu_sc as plsc

N = 10000
D = 128
E = 320000
G = 32

NC = 2
NS = 16
NW = NC * NS
CH = 64
J = 160
NP = 4
JP = J // NP
NB = 4
HP = JP // NB
E_PAD = NW * CH * J
ROWS_PER_TILE = 632
NPAD = NS * ROWS_PER_TILE

RBLK = 2000
GRID = N // RBLK


def _sc_body(hw_hbm, src_hbm, dst_hbm, zeros_hbm, out_hbm,
             srcv, dstv, rows_a, rows_b, rows_c, rows_d, agg,
             sem_a, sem_b, sem_c, sem_d):
    cid = lax.axis_index("c")
    sid = lax.axis_index("s")
    wid = sid * NC + cid
    bufs = ((rows_a, sem_a), (rows_b, sem_b), (rows_c, sem_c), (rows_d, sem_d))

    pltpu.sync_copy(zeros_hbm, agg.at[pl.ds(sid * ROWS_PER_TILE, ROWS_PER_TILE)])
    plsc.subcore_barrier()

    def _fill(buf, sem, j):
        pltpu.async_copy(hw_hbm.at[srcv.at[j]], buf, sem)

    def _drain(buf, sem):
        pltpu.make_async_copy(hw_hbm.at[pl.ds(0, CH)], buf, sem).wait()

    def _scatter(buf, j):
        pltpu.sync_copy(buf, agg.at[dstv.at[j]], add=True)

    for p in range(NP):
        pltpu.sync_copy(src_hbm.at[pl.ds(wid * J + p * JP, JP)], srcv)
        pltpu.sync_copy(dst_hbm.at[pl.ds(wid * J + p * JP, JP)], dstv)
        for k, (buf, sem) in enumerate(bufs):
            _fill(buf, sem, k)

        def step(i, carry):
            for k, (buf, sem) in enumerate(bufs):
                _drain(buf, sem)
                _scatter(buf, NB * i + k)

                @pl.when(i < HP - 1)
                def _next():
                    _fill(buf, sem, NB * i + NB + k)

            return carry

        lax.fori_loop(0, HP, step, 0)
    plsc.subcore_barrier()

    pltpu.sync_copy(agg.at[pl.ds(sid * ROWS_PER_TILE, ROWS_PER_TILE)],
                    out_hbm.at[cid, pl.ds(sid * ROWS_PER_TILE, ROWS_PER_TILE)])


_sc_scatter = functools.partial(
    pl.kernel,
    mesh=plsc.VectorSubcoreMesh(core_axis_name="c", subcore_axis_name="s",
                                num_cores=NC, num_subcores=NS),
    out_type=jax.ShapeDtypeStruct((NC, NPAD, D), jnp.float32),
    scratch_types=[
        pltpu.VMEM((JP, CH), jnp.int32),
        pltpu.VMEM((JP, CH), jnp.int32),
        pltpu.VMEM((CH, D), jnp.float32),
        pltpu.VMEM((CH, D), jnp.float32),
        pltpu.VMEM((CH, D), jnp.float32),
        pltpu.VMEM((CH, D), jnp.float32),
        pltpu.VMEM_SHARED((NPAD, D), jnp.float32),
        pltpu.SemaphoreType.DMA,
        pltpu.SemaphoreType.DMA,
        pltpu.SemaphoreType.DMA,
        pltpu.SemaphoreType.DMA,
    ],
)(_sc_body)


def _mm_body(x_ref, w_ref, o_ref):
    o_ref[...] = jnp.dot(x_ref[...], w_ref[...], preferred_element_type=jnp.float32)


_mm = pl.pallas_call(
    _mm_body,
    grid=(GRID,),
    in_specs=[
        pl.BlockSpec((RBLK, D), lambda i: (i, 0)),
        pl.BlockSpec((D, D), lambda i: (0, 0)),
    ],
    out_specs=pl.BlockSpec((RBLK, D), lambda i: (i, 0)),
    out_shape=jax.ShapeDtypeStruct((N, D), jnp.float32),
)


def _act_mm_body(p_ref, b_ref, w_ref, o_ref):
    h = jnp.maximum(p_ref[0] + p_ref[1] + b_ref[...], 0.0)
    o_ref[...] = jnp.dot(h, w_ref[...], preferred_element_type=jnp.float32)


_act_mm = pl.pallas_call(
    _act_mm_body,
    grid=(GRID,),
    in_specs=[
        pl.BlockSpec((NC, RBLK, D), lambda i: (0, i, 0)),
        pl.BlockSpec((1, D), lambda i: (0, 0)),
        pl.BlockSpec((D, D), lambda i: (0, 0)),
    ],
    out_specs=pl.BlockSpec((RBLK, D), lambda i: (i, 0)),
    out_shape=jax.ShapeDtypeStruct((N, D), jnp.float32),
)


def _pool_body(p_ref, b_ref, bat_ref, o_ref, sums, counts):
    i = pl.program_id(0)

    @pl.when(i == 0)
    def _init():
        sums[...] = jnp.zeros_like(sums)
        counts[...] = jnp.zeros_like(counts)

    h = jnp.maximum(p_ref[0] + p_ref[1] + b_ref[...], 0.0)
    onehot = (bat_ref[...] == lax.broadcasted_iota(jnp.int32, (1, G), 1)
              ).astype(jnp.float32)
    dn = (((0,), (0,)), ((), ()))
    sums[...] += lax.dot_general(onehot, h, dn, preferred_element_type=jnp.float32)
    counts[...] += lax.dot_general(onehot, jnp.ones_like(h), dn,
                                   preferred_element_type=jnp.float32)

    @pl.when(i == pl.num_programs(0) - 1)
    def _fin():
        o_ref[...] = sums[...] / jnp.maximum(counts[...], 1.0)


_pool = pl.pallas_call(
    _pool_body,
    grid=(GRID,),
    in_specs=[
        pl.BlockSpec((NC, RBLK, D), lambda i: (0, i, 0)),
        pl.BlockSpec((1, D), lambda i: (0, 0)),
        pl.BlockSpec((RBLK, 1), lambda i: (i, 0)),
    ],
    out_specs=pl.BlockSpec((G, D), lambda i: (0, 0)),
    out_shape=jax.ShapeDtypeStruct((G, D), jnp.float32),
    scratch_shapes=[
        pltpu.VMEM((G, D), jnp.float32),
        pltpu.VMEM((G, D), jnp.float32),
    ],
)


def kernel(x, edge_index, batch, W1, b1, W2, b2, W3, b3, W4, b4):
    src = edge_index[0].astype(jnp.int32)
    dst = edge_index[1].astype(jnp.int32)
    pad = E_PAD - E
    pad_src = jnp.arange(pad, dtype=jnp.int32) % N
    pad_dst = N + (jnp.arange(pad, dtype=jnp.int32) % (NPAD - N))
    src_r = jnp.concatenate([src, pad_src]).reshape(NW * J, CH)
    dst_r = jnp.concatenate([dst, pad_dst]).reshape(NW * J, CH)
    zeros = jnp.zeros((ROWS_PER_TILE, D), jnp.float32)
    bat = batch.astype(jnp.int32).reshape(N, 1)

    hw = _mm(x, W1)
    parts = _sc_scatter(hw, src_r, dst_r, zeros)
    for b, W in ((b1, W2), (b2, W3), (b3, W4)):
        hw = _act_mm(parts, b.reshape(1, D), W)
        parts = _sc_scatter(hw, src_r, dst_r, zeros)
    return _pool(parts, b4.reshape(1, D), bat)
